# baseline (speedup 1.0000x reference)
import functools

import jax
import jax.numpy as jnp
from jax import lax
from jax.experimental import pallas as pl
from jax.experimental.pallas import tpu as pltpu
from jax.experimental.pallas import tpu_sc as plsc

ROWS = 1024
COLS = 100000
TOPK = 5
MINACT = 0.5
LANES = 16
NCHAIN = 5
VPR = COLS // LANES
ITERS = VPR // NCHAIN
NEG = -3.0e38


def _insert(ts, x):
    t0, t1, t2, t3, t4 = ts
    y = jnp.minimum(t0, x)
    t0 = jnp.maximum(t0, x)
    y2 = jnp.minimum(t1, y)
    t1 = jnp.maximum(t1, y)
    y3 = jnp.minimum(t2, y2)
    t2 = jnp.maximum(t2, y2)
    y4 = jnp.minimum(t3, y3)
    t3 = jnp.maximum(t3, y3)
    t4 = jnp.maximum(t4, y4)
    return (t0, t1, t2, t3, t4)


def _permute(x, idx):
    dnums = lax.GatherDimensionNumbers(
        offset_dims=(), collapsed_slice_dims=(0,), start_index_map=(0,))
    return lax.gather(x, idx[:, None], dnums, slice_sizes=(1,),
                      mode=lax.GatherScatterMode.PROMISE_IN_BOUNDS)


def _lane_reduce_splat(x, op):
    iot = lax.iota(jnp.int32, LANES)
    for sh in (8, 4, 2, 1):
        x = op(x, _permute(x, iot ^ sh))
    return x


def _row_loss(ts):
    t0, t1, t2, t3, t4 = ts
    iot = lax.iota(jnp.int32, LANES)
    acc = jnp.zeros((LANES,), jnp.float32)
    for _ in range(TOPK):
        gs = _lane_reduce_splat(t0, jnp.maximum)
        acc = acc + gs
        cand = jnp.where(t0 == gs, iot, LANES)
        fs = _lane_reduce_splat(cand, jnp.minimum)
        pm = iot == fs
        t0 = jnp.where(pm, t1, t0)
        t1 = jnp.where(pm, t2, t1)
        t2 = jnp.where(pm, t3, t2)
        t3 = jnp.where(pm, t4, t3)
        t4 = jnp.where(pm, jnp.float32(NEG), t4)
    mean5 = acc * jnp.float32(1.0 / TOPK)
    return jnp.maximum(jnp.float32(MINACT) - mean5, 0.0)


CHUNK = COLS // 2
CVPR = CHUNK // LANES
CITERS = CVPR // NCHAIN


def _sc_body(x_hbm, out_hbm, buf0, buf1, part_v, sem0, sem1):
    c = lax.axis_index("c")
    s = lax.axis_index("s")
    wid = s * 2 + c
    rows_per_w = ROWS // 32
    row0 = wid * rows_per_w
    neg = jnp.full((LANES,), NEG, jnp.float32)

    def process_chunk(bref, chains):
        def scan_body(i, ch):
            base = i * (NCHAIN * LANES)
            out = []
            for j in range(NCHAIN):
                x = bref[pl.ds(base + j * LANES, LANES)]
                out.append(_insert(ch[j], x))
            return tuple(out)

        return lax.fori_loop(0, CITERS, scan_body, chains)

    pltpu.async_copy(x_hbm.at[pl.ds(row0 * COLS, CHUNK)], buf0, sem0)

    def row_body(r_local, part):
        r = row0 + r_local
        pltpu.async_copy(x_hbm.at[pl.ds(r * COLS + CHUNK, CHUNK)], buf1, sem1)
        pltpu.make_async_copy(
            x_hbm.at[pl.ds(r * COLS, CHUNK)], buf0, sem0).wait()

        init = tuple(tuple(neg for _ in range(TOPK)) for _ in range(NCHAIN))
        chains = process_chunk(buf0, init)

        rn = jnp.minimum(r + 1, row0 + rows_per_w - 1)
        pltpu.async_copy(x_hbm.at[pl.ds(rn * COLS, CHUNK)], buf0, sem0)

        pltpu.make_async_copy(
            x_hbm.at[pl.ds(r * COLS + CHUNK, CHUNK)], buf1, sem1).wait()
        chains = process_chunk(buf1, chains)

        merged = chains[0]
        for j in range(1, NCHAIN):
            for v in chains[j]:
                merged = _insert(merged, v)

        return part + _row_loss(merged)

    part = lax.fori_loop(0, rows_per_w, row_body,
                         jnp.zeros((LANES,), jnp.float32))
    pltpu.make_async_copy(
        x_hbm.at[pl.ds((row0 + rows_per_w - 1) * COLS, CHUNK)], buf0,
        sem0).wait()
    part_v[...] = part
    pltpu.sync_copy(part_v, out_hbm.at[pl.ds(wid * LANES, LANES)])


def _final_reduce_body(x_ref, o_ref):
    s = jnp.sum(x_ref[...]) * (1.0 / (LANES * ROWS))
    o_ref[...] = jnp.reshape(s, (1, 1))


def kernel(sparse_repr):
    mesh = plsc.VectorSubcoreMesh(core_axis_name="c", subcore_axis_name="s")
    sc_call = functools.partial(
        pl.kernel,
        mesh=mesh,
        out_type=jax.ShapeDtypeStruct((32 * LANES,), jnp.float32),
        scratch_types=[
            pltpu.VMEM((CHUNK,), jnp.float32),
            pltpu.VMEM((CHUNK,), jnp.float32),
            pltpu.VMEM((LANES,), jnp.float32),
            pltpu.SemaphoreType.DMA,
            pltpu.SemaphoreType.DMA,
        ],
    )(_sc_body)
    partials = sc_call(sparse_repr.reshape(-1))

    res = pl.pallas_call(
        _final_reduce_body,
        out_shape=jax.ShapeDtypeStruct((1, 1), jnp.float32),
    )(partials.reshape(1, 32 * LANES))
    return res[0, 0]


# speedup vs baseline: 1.6825x; 1.6825x over previous
import functools

import jax
import jax.numpy as jnp
from jax import lax
from jax.experimental import pallas as pl
from jax.experimental.pallas import tpu as pltpu
from jax.experimental.pallas import tpu_sc as plsc

ROWS = 1024
COLS = 100000
TOPK = 5
MINACT = 0.5
LANES = 16
NCHAIN = 5
VPR = COLS // LANES
ITERS = VPR // NCHAIN
NEG = -3.0e38


def _insert(ts, x):
    t0, t1, t2, t3, t4 = ts
    y = jnp.minimum(t0, x)
    t0 = jnp.maximum(t0, x)
    y2 = jnp.minimum(t1, y)
    t1 = jnp.maximum(t1, y)
    y3 = jnp.minimum(t2, y2)
    t2 = jnp.maximum(t2, y2)
    y4 = jnp.minimum(t3, y3)
    t3 = jnp.maximum(t3, y3)
    t4 = jnp.maximum(t4, y4)
    return (t0, t1, t2, t3, t4)


def _permute(x, idx):
    dnums = lax.GatherDimensionNumbers(
        offset_dims=(), collapsed_slice_dims=(0,), start_index_map=(0,))
    return lax.gather(x, idx[:, None], dnums, slice_sizes=(1,),
                      mode=lax.GatherScatterMode.PROMISE_IN_BOUNDS)


def _lane_reduce_splat(x, op):
    iot = lax.iota(jnp.int32, LANES)
    for sh in (8, 4, 2, 1):
        x = op(x, _permute(x, iot ^ sh))
    return x


def _row_loss(ts):
    t0, t1, t2, t3, t4 = ts
    iot = lax.iota(jnp.int32, LANES)
    acc = jnp.zeros((LANES,), jnp.float32)
    for _ in range(TOPK):
        gs = _lane_reduce_splat(t0, jnp.maximum)
        acc = acc + gs
        cand = jnp.where(t0 == gs, iot, LANES)
        fs = _lane_reduce_splat(cand, jnp.minimum)
        pm = iot == fs
        t0 = jnp.where(pm, t1, t0)
        t1 = jnp.where(pm, t2, t1)
        t2 = jnp.where(pm, t3, t2)
        t3 = jnp.where(pm, t4, t3)
        t4 = jnp.where(pm, jnp.float32(NEG), t4)
    mean5 = acc * jnp.float32(1.0 / TOPK)
    return jnp.maximum(jnp.float32(MINACT) - mean5, 0.0)


CHUNK0 = 49920
CHUNK1 = COLS - CHUNK0
CITERS0 = CHUNK0 // LANES // NCHAIN
CITERS1 = CHUNK1 // LANES // NCHAIN


def _sc_body(x_hbm, out_hbm, buf0, buf1, part_v, sem0, sem1):
    c = lax.axis_index("c")
    s = lax.axis_index("s")
    wid = s * 2 + c
    rows_per_w = ROWS // 32
    row0 = wid * rows_per_w
    neg = jnp.full((LANES,), NEG, jnp.float32)

    def process_chunk(bref, iters, chains):
        def scan_body(i, ch):
            base = i * (NCHAIN * LANES)
            out = []
            for j in range(NCHAIN):
                x = bref[pl.ds(base + j * LANES, LANES)]
                out.append(_insert(ch[j], x))
            return tuple(out)

        return lax.fori_loop(0, iters, scan_body, chains)

    pltpu.async_copy(x_hbm.at[row0, pl.ds(0, CHUNK0)], buf0, sem0)

    def row_body(r_local, part):
        r = row0 + r_local
        pltpu.async_copy(x_hbm.at[r, pl.ds(CHUNK0, CHUNK1)], buf1, sem1)
        pltpu.make_async_copy(
            x_hbm.at[r, pl.ds(0, CHUNK0)], buf0, sem0).wait()

        init = tuple(tuple(neg for _ in range(TOPK)) for _ in range(NCHAIN))
        chains = process_chunk(buf0, CITERS0, init)

        rn = jnp.minimum(r + 1, row0 + rows_per_w - 1)
        pltpu.async_copy(x_hbm.at[rn, pl.ds(0, CHUNK0)], buf0, sem0)

        pltpu.make_async_copy(
            x_hbm.at[r, pl.ds(CHUNK0, CHUNK1)], buf1, sem1).wait()
        chains = process_chunk(buf1, CITERS1, chains)

        merged = chains[0]
        for j in range(1, NCHAIN):
            for v in chains[j]:
                merged = _insert(merged, v)

        return part + _row_loss(merged)

    part = lax.fori_loop(0, rows_per_w, row_body,
                         jnp.zeros((LANES,), jnp.float32))
    pltpu.make_async_copy(
        x_hbm.at[row0 + rows_per_w - 1, pl.ds(0, CHUNK0)], buf0,
        sem0).wait()
    part_v[...] = part
    pltpu.sync_copy(part_v, out_hbm.at[pl.ds(wid * LANES, LANES)])


def _final_reduce_body(x_ref, o_ref):
    s = jnp.sum(x_ref[...]) * (1.0 / (LANES * ROWS))
    o_ref[...] = jnp.reshape(s, (1, 1))


def kernel(sparse_repr):
    mesh = plsc.VectorSubcoreMesh(core_axis_name="c", subcore_axis_name="s")
    sc_call = functools.partial(
        pl.kernel,
        mesh=mesh,
        out_type=jax.ShapeDtypeStruct((32 * LANES,), jnp.float32),
        scratch_types=[
            pltpu.VMEM((CHUNK0,), jnp.float32),
            pltpu.VMEM((CHUNK1,), jnp.float32),
            pltpu.VMEM((LANES,), jnp.float32),
            pltpu.SemaphoreType.DMA,
            pltpu.SemaphoreType.DMA,
        ],
    )(_sc_body)
    partials = sc_call(sparse_repr)

    res = pl.pallas_call(
        _final_reduce_body,
        out_shape=jax.ShapeDtypeStruct((1, 1), jnp.float32),
    )(partials.reshape(1, 32 * LANES))
    return res[0, 0]


# speedup vs baseline: 1.7071x; 1.0146x over previous
import functools

import jax
import jax.numpy as jnp
from jax import lax
from jax.experimental import pallas as pl
from jax.experimental import pallas as pl
from jax.experimental.pallas import tpu as pltpu
from jax.experimental.pallas import tpu_sc as plsc

ROWS = 1024
COLS = 100000
TOPK = 5
MINACT = 0.5
LANES = 16
NEG = -3.0e38

CHUNK0 = 49920
CHUNK1 = COLS - CHUNK0
BLOCK = 40
NB0 = CHUNK0 // LANES // BLOCK
NB1 = (CHUNK1 // LANES) // BLOCK
TAIL1 = CHUNK1 // LANES - NB1 * BLOCK


def _insert(ts, x):
    t0, t1, t2, t3, t4 = ts
    y = jnp.minimum(t0, x)
    t0 = jnp.maximum(t0, x)
    y2 = jnp.minimum(t1, y)
    t1 = jnp.maximum(t1, y)
    y3 = jnp.minimum(t2, y2)
    t2 = jnp.maximum(t2, y2)
    y4 = jnp.minimum(t3, y3)
    t3 = jnp.maximum(t3, y3)
    t4 = jnp.maximum(t4, y4)
    return (t0, t1, t2, t3, t4)


def _permute(x, idx):
    dnums = lax.GatherDimensionNumbers(
        offset_dims=(), collapsed_slice_dims=(0,), start_index_map=(0,))
    return lax.gather(x, idx[:, None], dnums, slice_sizes=(1,),
                      mode=lax.GatherScatterMode.PROMISE_IN_BOUNDS)


def _lane_reduce_splat(x, op):
    iot = lax.iota(jnp.int32, LANES)
    for sh in (8, 4, 2, 1):
        x = op(x, _permute(x, iot ^ sh))
    return x


def _cmpex(x, j, want_min):
    iot = lax.iota(jnp.int32, LANES)
    p = _permute(x, iot ^ j)
    return jnp.where(want_min, jnp.minimum(x, p), jnp.maximum(x, p))


def _sort_desc(x):
    iot = lax.iota(jnp.int32, LANES)
    for k in (2, 4, 8, 16):
        lk = k.bit_length() - 1
        j = k >> 1
        while j:
            lj = j.bit_length() - 1
            want = ((iot >> lj) ^ (iot >> lk)) & 1
            x = _cmpex(x, j, want == 1)
            j >>= 1
    return x


def _resort_asc(x):
    iot = lax.iota(jnp.int32, LANES)
    for j in (8, 4, 2, 1):
        x = _cmpex(x, j, (iot & j) == 0)
    return x


def _merge_into_c(c_asc, v):
    h = jnp.maximum(c_asc, _sort_desc(v))
    return _resort_asc(h)


def _block(bref, c_v, thr_v, base, nv):
    nch = nv // 5
    ms = []
    for ch in range(nch):
        m = bref[pl.ds(base + ch * 5 * LANES, LANES)]
        for j in range(1, 5):
            m = jnp.maximum(m, bref[pl.ds(base + (ch * 5 + j) * LANES, LANES)])
        ms.append(m)
    while len(ms) > 1:
        ms = [jnp.maximum(ms[i], ms[i + 1]) if i + 1 < len(ms) else ms[i]
              for i in range(0, len(ms), 2)]
    bmax = _lane_reduce_splat(ms[0], jnp.maximum)
    pred = bmax[0] > thr_v[...][0]

    @pl.when(pred)
    def hit():
        c_asc = c_v[...]
        neg = jnp.full((LANES,), NEG, jnp.float32)
        tsa = (neg,) * 5
        tsb = (neg,) * 5
        half = (nch // 2) * 5
        for v in range(half):
            tsa = _insert(tsa, bref[pl.ds(base + v * LANES, LANES)])
        for v in range(half, nv):
            tsb = _insert(tsb, bref[pl.ds(base + v * LANES, LANES)])
        for v in tsb:
            tsa = _insert(tsa, v)
        for v in tsa:
            c_asc = _merge_into_c(c_asc, v)
        c_v[...] = c_asc
        thr_v[...] = _permute(c_asc, jnp.full((LANES,), 11, jnp.int32))


def _row_loss(c_asc):
    iot = lax.iota(jnp.int32, LANES)
    masked = jnp.where(iot >= LANES - TOPK, c_asc, 0.0)
    s = _lane_reduce_splat(masked, jnp.add)
    mean5 = s * jnp.float32(1.0 / TOPK)
    return jnp.maximum(jnp.float32(MINACT) - mean5, 0.0)


def _sc_body(x_hbm, out_hbm, buf0, buf1, part_v, c_v, thr_v, sem0, sem1):
    c = lax.axis_index("c")
    s = lax.axis_index("s")
    wid = s * 2 + c
    rows_per_w = ROWS // 32
    row0 = wid * rows_per_w

    def process_chunk(bref, nblocks, tail):
        def blk(i, _):
            _block(bref, c_v, thr_v, i * BLOCK * LANES, BLOCK)
            return 0

        lax.fori_loop(0, nblocks, blk, 0)
        if tail:
            _block(bref, c_v, thr_v, nblocks * BLOCK * LANES, tail)

    pltpu.async_copy(x_hbm.at[row0, pl.ds(0, CHUNK0)], buf0, sem0)

    def row_body(r_local, part):
        r = row0 + r_local
        pltpu.async_copy(x_hbm.at[r, pl.ds(CHUNK0, CHUNK1)], buf1, sem1)
        pltpu.make_async_copy(
            x_hbm.at[r, pl.ds(0, CHUNK0)], buf0, sem0).wait()

        c_v[...] = jnp.full((LANES,), NEG, jnp.float32)
        thr_v[...] = jnp.full((LANES,), -jnp.inf, jnp.float32)
        process_chunk(buf0, NB0, 0)

        rn = jnp.minimum(r + 1, row0 + rows_per_w - 1)
        pltpu.async_copy(x_hbm.at[rn, pl.ds(0, CHUNK0)], buf0, sem0)

        pltpu.make_async_copy(
            x_hbm.at[r, pl.ds(CHUNK0, CHUNK1)], buf1, sem1).wait()
        process_chunk(buf1, NB1, TAIL1)

        return part + _row_loss(c_v[...])

    part = lax.fori_loop(0, rows_per_w, row_body,
                         jnp.zeros((LANES,), jnp.float32))
    pltpu.make_async_copy(
        x_hbm.at[row0 + rows_per_w - 1, pl.ds(0, CHUNK0)], buf0,
        sem0).wait()
    part_v[...] = part
    pltpu.sync_copy(part_v, out_hbm.at[pl.ds(wid * LANES, LANES)])


def _final_reduce_body(x_ref, o_ref):
    s = jnp.sum(x_ref[...]) * (1.0 / (LANES * ROWS))
    o_ref[...] = jnp.reshape(s, (1, 1))


def kernel(sparse_repr):
    mesh = plsc.VectorSubcoreMesh(core_axis_name="c", subcore_axis_name="s")
    sc_call = functools.partial(
        pl.kernel,
        mesh=mesh,
        out_type=jax.ShapeDtypeStruct((32 * LANES,), jnp.float32),
        scratch_types=[
            pltpu.VMEM((CHUNK0,), jnp.float32),
            pltpu.VMEM((CHUNK1,), jnp.float32),
            pltpu.VMEM((LANES,), jnp.float32),
            pltpu.VMEM((LANES,), jnp.float32),
            pltpu.VMEM((LANES,), jnp.float32),
            pltpu.SemaphoreType.DMA,
            pltpu.SemaphoreType.DMA,
        ],
    )(_sc_body)
    partials = sc_call(sparse_repr)

    res = pl.pallas_call(
        _final_reduce_body,
        out_shape=jax.ShapeDtypeStruct((1, 1), jnp.float32),
    )(partials.reshape(1, 32 * LANES))
    return res[0, 0]
